# baseline (speedup 1.0000x reference)
import functools

import jax
import jax.numpy as jnp
from jax import lax
from jax.experimental import pallas as pl
from jax.experimental.pallas import tpu as pltpu
from jax.experimental.pallas import tpu_sc as plsc

EPS = 1e-12
NUM_SC_CORES = 2
NUM_SC_SUBCORES = 16
NW = NUM_SC_CORES * NUM_SC_SUBCORES
GATHER_CHUNK = 64
TOK_BLOCK = 256
N_SPLIT = 1


def _sc_gather(word_emb, flat_ids):
    n_tok = flat_ids.shape[0]
    hidden = word_emb.shape[1]
    per_w = n_tok // NW
    n_chunks = per_w // GATHER_CHUNK
    mesh = plsc.VectorSubcoreMesh(core_axis_name="c", subcore_axis_name="s")

    @functools.partial(
        pl.kernel,
        out_type=jax.ShapeDtypeStruct((n_tok, hidden), jnp.float32),
        mesh=mesh,
        scratch_types=[
            pltpu.VMEM((per_w,), jnp.int32),
            pltpu.VMEM((2, GATHER_CHUNK, hidden), jnp.float32),
            pltpu.SemaphoreType.DMA,
            pltpu.SemaphoreType.DMA,
            pltpu.SemaphoreType.DMA,
            pltpu.SemaphoreType.DMA,
        ],
    )
    def gather_kernel(table_hbm, idx_hbm, out_hbm, idx_v, rows_v, g0, g1, w0, w1):
        gsems = (g0, g1)
        wsems = (w0, w1)
        wid = lax.axis_index("s") * NUM_SC_CORES + lax.axis_index("c")
        base = wid * per_w
        pltpu.sync_copy(idx_hbm.at[pl.ds(base, per_w)], idx_v)
        gathers = [None] * n_chunks
        writes = [None] * n_chunks
        for c in range(n_chunks):
            if c >= 2 and writes[c - 2] is not None:
                writes[c - 2].wait()
            gathers[c] = pltpu.async_copy(
                table_hbm.at[idx_v.at[pl.ds(c * GATHER_CHUNK, GATHER_CHUNK)]],
                rows_v.at[c % 2],
                gsems[c % 2],
            )
            if c >= 1:
                gathers[c - 1].wait()
                writes[c - 1] = pltpu.async_copy(
                    rows_v.at[(c - 1) % 2],
                    out_hbm.at[pl.ds(base + (c - 1) * GATHER_CHUNK, GATHER_CHUNK)],
                    wsems[(c - 1) % 2],
                )
        gathers[n_chunks - 1].wait()
        pltpu.sync_copy(
            rows_v.at[(n_chunks - 1) % 2],
            out_hbm.at[pl.ds(base + (n_chunks - 1) * GATHER_CHUNK, GATHER_CHUNK)],
        )
        if n_chunks >= 2 and writes[n_chunks - 2] is not None:
            writes[n_chunks - 2].wait()

    return gather_kernel(word_emb, flat_ids)


def _tc_add_ln(gathered, pos_block, type_emb, tt_col, ln_w2, ln_b2, s_base, seq):
    n_tok, hidden = gathered.shape
    blocks_per_seq = seq // TOK_BLOCK
    n_batch = n_tok // seq
    base_block = s_base // TOK_BLOCK

    def tok_block(i, j):
        return (j * blocks_per_seq + i, 0)

    def body(g_ref, pos_ref, type_ref, tt_ref, w_ref, b_ref, o_ref):
        t0 = type_ref[0:1, :]
        dt = type_ref[1:2, :] - t0
        e = (
            g_ref[...]
            + pos_ref[...]
            + t0
            + tt_ref[...].astype(jnp.float32) * dt
        )
        mean = jnp.mean(e, axis=1, keepdims=True)
        ec = e - mean
        var = jnp.mean(ec * ec, axis=1, keepdims=True)
        o_ref[...] = ec * lax.rsqrt(var + EPS) * w_ref[...] + b_ref[...]

    return pl.pallas_call(
        body,
        grid=(blocks_per_seq, n_batch),
        in_specs=[
            pl.BlockSpec((TOK_BLOCK, hidden), tok_block),
            pl.BlockSpec(
                (TOK_BLOCK, hidden),
                lambda i, j: (
                    lax.rem(i + base_block, blocks_per_seq),
                    0,
                ),
            ),
            pl.BlockSpec((2, hidden), lambda i, j: (0, 0)),
            pl.BlockSpec((TOK_BLOCK, 1), tok_block),
            pl.BlockSpec((1, hidden), lambda i, j: (0, 0)),
            pl.BlockSpec((1, hidden), lambda i, j: (0, 0)),
        ],
        out_specs=pl.BlockSpec((TOK_BLOCK, hidden), tok_block),
        out_shape=jax.ShapeDtypeStruct((n_tok, hidden), jnp.float32),
    )(gathered, pos_block, type_emb, tt_col, ln_w2, ln_b2)


def kernel(input_ids, token_type_ids, word_emb, pos_emb, type_emb, ln_w, ln_b):
    b, s = input_ids.shape
    hidden = word_emb.shape[1]
    n_tok = b * s
    flat_ids = input_ids.reshape(-1)
    tt_col = token_type_ids.reshape(-1, 1)
    pos_block = pos_emb[:s]
    ln_w2 = ln_w.reshape(1, hidden)
    ln_b2 = ln_b.reshape(1, hidden)

    piece = n_tok // N_SPLIT
    outs = []
    for p in range(N_SPLIT):
        lo = p * piece
        gathered = _sc_gather(word_emb, flat_ids[lo : lo + piece])
        outs.append(
            _tc_add_ln(
                gathered,
                pos_block,
                type_emb,
                tt_col[lo : lo + piece],
                ln_w2,
                ln_b2,
                lo % s,
                s,
            )
        )
    out = jnp.concatenate(outs, axis=0) if N_SPLIT > 1 else outs[0]
    return out.reshape(b, s, hidden)


# speedup vs baseline: 1.1516x; 1.1516x over previous
import functools

import jax
import jax.numpy as jnp
from jax import lax
from jax.experimental import pallas as pl
from jax.experimental.pallas import tpu as pltpu
from jax.experimental.pallas import tpu_sc as plsc

EPS = 1e-12
NUM_SC_CORES = 2
NUM_SC_SUBCORES = 16
NW = NUM_SC_CORES * NUM_SC_SUBCORES
GATHER_CHUNK = 64
TOK_BLOCK = 512
N_SPLIT = 1


def _sc_gather(word_emb, flat_ids):
    n_tok = flat_ids.shape[0]
    hidden = word_emb.shape[1]
    per_w = n_tok // NW
    n_chunks = per_w // GATHER_CHUNK
    mesh = plsc.VectorSubcoreMesh(core_axis_name="c", subcore_axis_name="s")

    @functools.partial(
        pl.kernel,
        out_type=jax.ShapeDtypeStruct((n_tok, hidden), jnp.float32),
        mesh=mesh,
        scratch_types=[
            pltpu.VMEM((per_w,), jnp.int32),
            pltpu.VMEM((2, GATHER_CHUNK, hidden), jnp.float32),
            pltpu.SemaphoreType.DMA,
            pltpu.SemaphoreType.DMA,
            pltpu.SemaphoreType.DMA,
            pltpu.SemaphoreType.DMA,
        ],
    )
    def gather_kernel(table_hbm, idx_hbm, out_hbm, idx_v, rows_v, g0, g1, w0, w1):
        gsems = (g0, g1)
        wsems = (w0, w1)
        wid = lax.axis_index("s") * NUM_SC_CORES + lax.axis_index("c")
        base = wid * per_w
        pltpu.sync_copy(idx_hbm.at[pl.ds(base, per_w)], idx_v)
        gathers = [None] * n_chunks
        writes = [None] * n_chunks
        for c in range(n_chunks):
            if c >= 2 and writes[c - 2] is not None:
                writes[c - 2].wait()
            gathers[c] = pltpu.async_copy(
                table_hbm.at[idx_v.at[pl.ds(c * GATHER_CHUNK, GATHER_CHUNK)]],
                rows_v.at[c % 2],
                gsems[c % 2],
            )
            if c >= 1:
                gathers[c - 1].wait()
                writes[c - 1] = pltpu.async_copy(
                    rows_v.at[(c - 1) % 2],
                    out_hbm.at[pl.ds(base + (c - 1) * GATHER_CHUNK, GATHER_CHUNK)],
                    wsems[(c - 1) % 2],
                )
        gathers[n_chunks - 1].wait()
        pltpu.sync_copy(
            rows_v.at[(n_chunks - 1) % 2],
            out_hbm.at[pl.ds(base + (n_chunks - 1) * GATHER_CHUNK, GATHER_CHUNK)],
        )
        if n_chunks >= 2 and writes[n_chunks - 2] is not None:
            writes[n_chunks - 2].wait()

    return gather_kernel(word_emb, flat_ids)


def _tc_add_ln(gathered, pos_block, type_emb, tt_col, ln_w2, ln_b2, s_base, seq):
    n_tok, hidden = gathered.shape
    blocks_per_seq = seq // TOK_BLOCK
    n_batch = n_tok // seq
    base_block = s_base // TOK_BLOCK

    def tok_block(i, j):
        return (j * blocks_per_seq + i, 0)

    def body(g_ref, pos_ref, type_ref, tt_ref, w_ref, b_ref, o_ref):
        t0 = type_ref[0:1, :]
        dt = type_ref[1:2, :] - t0
        e = (
            g_ref[...]
            + pos_ref[...]
            + t0
            + tt_ref[...].astype(jnp.float32) * dt
        )
        mean = jnp.mean(e, axis=1, keepdims=True)
        ec = e - mean
        var = jnp.mean(ec * ec, axis=1, keepdims=True)
        o_ref[...] = ec * lax.rsqrt(var + EPS) * w_ref[...] + b_ref[...]

    return pl.pallas_call(
        body,
        grid=(blocks_per_seq, n_batch),
        in_specs=[
            pl.BlockSpec((TOK_BLOCK, hidden), tok_block),
            pl.BlockSpec(
                (TOK_BLOCK, hidden),
                lambda i, j: (
                    lax.rem(i + base_block, blocks_per_seq),
                    0,
                ),
            ),
            pl.BlockSpec((2, hidden), lambda i, j: (0, 0)),
            pl.BlockSpec((TOK_BLOCK, 1), tok_block),
            pl.BlockSpec((1, hidden), lambda i, j: (0, 0)),
            pl.BlockSpec((1, hidden), lambda i, j: (0, 0)),
        ],
        out_specs=pl.BlockSpec((TOK_BLOCK, hidden), tok_block),
        out_shape=jax.ShapeDtypeStruct((n_tok, hidden), jnp.float32),
    )(gathered, pos_block, type_emb, tt_col, ln_w2, ln_b2)


def kernel(input_ids, token_type_ids, word_emb, pos_emb, type_emb, ln_w, ln_b):
    b, s = input_ids.shape
    hidden = word_emb.shape[1]
    n_tok = b * s
    flat_ids = input_ids.reshape(-1)
    tt_col = token_type_ids.reshape(-1, 1)
    pos_block = pos_emb[:s]
    ln_w2 = ln_w.reshape(1, hidden)
    ln_b2 = ln_b.reshape(1, hidden)

    piece = n_tok // N_SPLIT
    outs = []
    for p in range(N_SPLIT):
        lo = p * piece
        gathered = _sc_gather(word_emb, flat_ids[lo : lo + piece])
        outs.append(
            _tc_add_ln(
                gathered,
                pos_block,
                type_emb,
                tt_col[lo : lo + piece],
                ln_w2,
                ln_b2,
                lo % s,
                s,
            )
        )
    out = jnp.concatenate(outs, axis=0) if N_SPLIT > 1 else outs[0]
    return out.reshape(b, s, hidden)


# speedup vs baseline: 1.2291x; 1.0672x over previous
import functools

import jax
import jax.numpy as jnp
from jax import lax
from jax.experimental import pallas as pl
from jax.experimental.pallas import tpu as pltpu
from jax.experimental.pallas import tpu_sc as plsc

EPS = 1e-12
NUM_SC_CORES = 2
NUM_SC_SUBCORES = 16
NW = NUM_SC_CORES * NUM_SC_SUBCORES
GATHER_CHUNK = 64
TOK_BLOCK = 1024
N_SPLIT = 1


def _sc_gather(word_emb, flat_ids):
    n_tok = flat_ids.shape[0]
    hidden = word_emb.shape[1]
    per_w = n_tok // NW
    n_chunks = per_w // GATHER_CHUNK
    mesh = plsc.VectorSubcoreMesh(core_axis_name="c", subcore_axis_name="s")

    @functools.partial(
        pl.kernel,
        out_type=jax.ShapeDtypeStruct((n_tok, hidden), jnp.float32),
        mesh=mesh,
        scratch_types=[
            pltpu.VMEM((per_w,), jnp.int32),
            pltpu.VMEM((2, GATHER_CHUNK, hidden), jnp.float32),
            pltpu.SemaphoreType.DMA,
            pltpu.SemaphoreType.DMA,
            pltpu.SemaphoreType.DMA,
            pltpu.SemaphoreType.DMA,
        ],
    )
    def gather_kernel(table_hbm, idx_hbm, out_hbm, idx_v, rows_v, g0, g1, w0, w1):
        gsems = (g0, g1)
        wsems = (w0, w1)
        wid = lax.axis_index("s") * NUM_SC_CORES + lax.axis_index("c")
        base = wid * per_w
        pltpu.sync_copy(idx_hbm.at[pl.ds(base, per_w)], idx_v)
        gathers = [None] * n_chunks
        writes = [None] * n_chunks
        for c in range(n_chunks):
            if c >= 2 and writes[c - 2] is not None:
                writes[c - 2].wait()
            gathers[c] = pltpu.async_copy(
                table_hbm.at[idx_v.at[pl.ds(c * GATHER_CHUNK, GATHER_CHUNK)]],
                rows_v.at[c % 2],
                gsems[c % 2],
            )
            if c >= 1:
                gathers[c - 1].wait()
                writes[c - 1] = pltpu.async_copy(
                    rows_v.at[(c - 1) % 2],
                    out_hbm.at[pl.ds(base + (c - 1) * GATHER_CHUNK, GATHER_CHUNK)],
                    wsems[(c - 1) % 2],
                )
        gathers[n_chunks - 1].wait()
        pltpu.sync_copy(
            rows_v.at[(n_chunks - 1) % 2],
            out_hbm.at[pl.ds(base + (n_chunks - 1) * GATHER_CHUNK, GATHER_CHUNK)],
        )
        if n_chunks >= 2 and writes[n_chunks - 2] is not None:
            writes[n_chunks - 2].wait()

    return gather_kernel(word_emb, flat_ids)


def _tc_add_ln(gathered, pos_block, type_emb, tt_col, ln_w2, ln_b2, s_base, seq):
    n_tok, hidden = gathered.shape
    blocks_per_seq = seq // TOK_BLOCK
    n_batch = n_tok // seq
    base_block = s_base // TOK_BLOCK

    def tok_block(i, j):
        return (j * blocks_per_seq + i, 0)

    def body(g_ref, pos_ref, type_ref, tt_ref, w_ref, b_ref, o_ref):
        t0 = type_ref[0:1, :]
        dt = type_ref[1:2, :] - t0
        e = (
            g_ref[...]
            + pos_ref[...]
            + t0
            + tt_ref[...].astype(jnp.float32) * dt
        )
        mean = jnp.mean(e, axis=1, keepdims=True)
        ec = e - mean
        var = jnp.mean(ec * ec, axis=1, keepdims=True)
        o_ref[...] = ec * lax.rsqrt(var + EPS) * w_ref[...] + b_ref[...]

    return pl.pallas_call(
        body,
        grid=(blocks_per_seq, n_batch),
        in_specs=[
            pl.BlockSpec((TOK_BLOCK, hidden), tok_block),
            pl.BlockSpec(
                (TOK_BLOCK, hidden),
                lambda i, j: (
                    lax.rem(i + base_block, blocks_per_seq),
                    0,
                ),
            ),
            pl.BlockSpec((2, hidden), lambda i, j: (0, 0)),
            pl.BlockSpec((TOK_BLOCK, 1), tok_block),
            pl.BlockSpec((1, hidden), lambda i, j: (0, 0)),
            pl.BlockSpec((1, hidden), lambda i, j: (0, 0)),
        ],
        out_specs=pl.BlockSpec((TOK_BLOCK, hidden), tok_block),
        out_shape=jax.ShapeDtypeStruct((n_tok, hidden), jnp.float32),
    )(gathered, pos_block, type_emb, tt_col, ln_w2, ln_b2)


def kernel(input_ids, token_type_ids, word_emb, pos_emb, type_emb, ln_w, ln_b):
    b, s = input_ids.shape
    hidden = word_emb.shape[1]
    n_tok = b * s
    flat_ids = input_ids.reshape(-1)
    tt_col = token_type_ids.reshape(-1, 1)
    pos_block = pos_emb[:s]
    ln_w2 = ln_w.reshape(1, hidden)
    ln_b2 = ln_b.reshape(1, hidden)

    piece = n_tok // N_SPLIT
    outs = []
    for p in range(N_SPLIT):
        lo = p * piece
        gathered = _sc_gather(word_emb, flat_ids[lo : lo + piece])
        outs.append(
            _tc_add_ln(
                gathered,
                pos_block,
                type_emb,
                tt_col[lo : lo + piece],
                ln_w2,
                ln_b2,
                lo % s,
                s,
            )
        )
    out = jnp.concatenate(outs, axis=0) if N_SPLIT > 1 else outs[0]
    return out.reshape(b, s, hidden)


# speedup vs baseline: 1.2495x; 1.0166x over previous
import functools

import jax
import jax.numpy as jnp
from jax import lax
from jax.experimental import pallas as pl
from jax.experimental.pallas import tpu as pltpu
from jax.experimental.pallas import tpu_sc as plsc

EPS = 1e-12
NUM_SC_CORES = 2
NUM_SC_SUBCORES = 16
NW = NUM_SC_CORES * NUM_SC_SUBCORES
GATHER_CHUNK = 64
TOK_BLOCK = 2048
N_SPLIT = 1


def _sc_gather(word_emb, flat_ids):
    n_tok = flat_ids.shape[0]
    hidden = word_emb.shape[1]
    per_w = n_tok // NW
    n_chunks = per_w // GATHER_CHUNK
    mesh = plsc.VectorSubcoreMesh(core_axis_name="c", subcore_axis_name="s")

    @functools.partial(
        pl.kernel,
        out_type=jax.ShapeDtypeStruct((n_tok, hidden), jnp.float32),
        mesh=mesh,
        scratch_types=[
            pltpu.VMEM((per_w,), jnp.int32),
            pltpu.VMEM((2, GATHER_CHUNK, hidden), jnp.float32),
            pltpu.SemaphoreType.DMA,
            pltpu.SemaphoreType.DMA,
            pltpu.SemaphoreType.DMA,
            pltpu.SemaphoreType.DMA,
        ],
    )
    def gather_kernel(table_hbm, idx_hbm, out_hbm, idx_v, rows_v, g0, g1, w0, w1):
        gsems = (g0, g1)
        wsems = (w0, w1)
        wid = lax.axis_index("s") * NUM_SC_CORES + lax.axis_index("c")
        base = wid * per_w
        pltpu.sync_copy(idx_hbm.at[pl.ds(base, per_w)], idx_v)
        gathers = [None] * n_chunks
        writes = [None] * n_chunks
        for c in range(n_chunks):
            if c >= 2 and writes[c - 2] is not None:
                writes[c - 2].wait()
            gathers[c] = pltpu.async_copy(
                table_hbm.at[idx_v.at[pl.ds(c * GATHER_CHUNK, GATHER_CHUNK)]],
                rows_v.at[c % 2],
                gsems[c % 2],
            )
            if c >= 1:
                gathers[c - 1].wait()
                writes[c - 1] = pltpu.async_copy(
                    rows_v.at[(c - 1) % 2],
                    out_hbm.at[pl.ds(base + (c - 1) * GATHER_CHUNK, GATHER_CHUNK)],
                    wsems[(c - 1) % 2],
                )
        gathers[n_chunks - 1].wait()
        pltpu.sync_copy(
            rows_v.at[(n_chunks - 1) % 2],
            out_hbm.at[pl.ds(base + (n_chunks - 1) * GATHER_CHUNK, GATHER_CHUNK)],
        )
        if n_chunks >= 2 and writes[n_chunks - 2] is not None:
            writes[n_chunks - 2].wait()

    return gather_kernel(word_emb, flat_ids)


def _tc_add_ln(gathered, pos_block, type_emb, tt_col, ln_w2, ln_b2, s_base, seq):
    n_tok, hidden = gathered.shape
    blocks_per_seq = seq // TOK_BLOCK
    n_batch = n_tok // seq
    base_block = s_base // TOK_BLOCK

    def tok_block(i, j):
        return (j * blocks_per_seq + i, 0)

    def body(g_ref, pos_ref, type_ref, tt_ref, w_ref, b_ref, o_ref):
        t0 = type_ref[0:1, :]
        dt = type_ref[1:2, :] - t0
        e = (
            g_ref[...]
            + pos_ref[...]
            + t0
            + tt_ref[...].astype(jnp.float32) * dt
        )
        mean = jnp.mean(e, axis=1, keepdims=True)
        ec = e - mean
        var = jnp.mean(ec * ec, axis=1, keepdims=True)
        o_ref[...] = ec * lax.rsqrt(var + EPS) * w_ref[...] + b_ref[...]

    return pl.pallas_call(
        body,
        grid=(blocks_per_seq, n_batch),
        in_specs=[
            pl.BlockSpec((TOK_BLOCK, hidden), tok_block),
            pl.BlockSpec(
                (TOK_BLOCK, hidden),
                lambda i, j: (
                    lax.rem(i + base_block, blocks_per_seq),
                    0,
                ),
            ),
            pl.BlockSpec((2, hidden), lambda i, j: (0, 0)),
            pl.BlockSpec((TOK_BLOCK, 1), tok_block),
            pl.BlockSpec((1, hidden), lambda i, j: (0, 0)),
            pl.BlockSpec((1, hidden), lambda i, j: (0, 0)),
        ],
        out_specs=pl.BlockSpec((TOK_BLOCK, hidden), tok_block),
        out_shape=jax.ShapeDtypeStruct((n_tok, hidden), jnp.float32),
    )(gathered, pos_block, type_emb, tt_col, ln_w2, ln_b2)


def kernel(input_ids, token_type_ids, word_emb, pos_emb, type_emb, ln_w, ln_b):
    b, s = input_ids.shape
    hidden = word_emb.shape[1]
    n_tok = b * s
    flat_ids = input_ids.reshape(-1)
    tt_col = token_type_ids.reshape(-1, 1)
    pos_block = pos_emb[:s]
    ln_w2 = ln_w.reshape(1, hidden)
    ln_b2 = ln_b.reshape(1, hidden)

    piece = n_tok // N_SPLIT
    outs = []
    for p in range(N_SPLIT):
        lo = p * piece
        gathered = _sc_gather(word_emb, flat_ids[lo : lo + piece])
        outs.append(
            _tc_add_ln(
                gathered,
                pos_block,
                type_emb,
                tt_col[lo : lo + piece],
                ln_w2,
                ln_b2,
                lo % s,
                s,
            )
        )
    out = jnp.concatenate(outs, axis=0) if N_SPLIT > 1 else outs[0]
    return out.reshape(b, s, hidden)
